# baseline (speedup 1.0000x reference)
import jax
import jax.numpy as jnp
from jax import lax
from jax.experimental import pallas as pl
from jax.experimental.pallas import tpu as pltpu
from jax.experimental.pallas import tpu_sc as plsc

B = 16384
D = 32
L = 16
NC = 2
NS = 16
NW = NC * NS
BPW = B // NW
CHUNK = 128
NCH = BPW // CHUNK
GROUPS = BPW // L


def _sc_body(uid_hbm, mid_hbm, uemb_hbm, memb_hbm, ubias_hbm, mbias_hbm,
             out_hbm, uidx_v, midx_v, urows_v, mrows_v, ubias_v, mbias_v,
             out_v, sem):
    c = lax.axis_index("c")
    s = lax.axis_index("s")
    wid = s * NC + c
    base = wid * BPW

    cp_u = pltpu.async_copy(uid_hbm.at[pl.ds(base, BPW)], uidx_v, sem)
    cp_m = pltpu.async_copy(mid_hbm.at[pl.ds(base, BPW)], midx_v, sem)
    cp_u.wait()
    cp_m.wait()

    pend = []
    for j in range(NCH):
        sl = pl.ds(j * CHUNK, CHUNK)
        idx_u = uidx_v.at[sl]
        idx_m = midx_v.at[sl]
        pend.append(pltpu.async_copy(uemb_hbm.at[idx_u], urows_v.at[sl], sem))
        pend.append(pltpu.async_copy(memb_hbm.at[idx_m], mrows_v.at[sl], sem))
        pend.append(pltpu.async_copy(ubias_hbm.at[idx_u], ubias_v.at[sl], sem))
        pend.append(pltpu.async_copy(mbias_hbm.at[idx_m], mbias_v.at[sl], sem))
    for cp in pend:
        cp.wait()

    def group_body(g, carry):
        rows = g * L + lax.iota(jnp.int32, L)
        acc = ubias_v[pl.ds(g * L, L)] + mbias_v[pl.ds(g * L, L)]
        for d in range(D):
            col = jnp.full((L,), d, jnp.int32)
            uu = plsc.load_gather(urows_v, [rows, col])
            mm = plsc.load_gather(mrows_v, [rows, col])
            acc = acc + uu * mm
        out_v[pl.ds(g * L, L)] = acc
        return carry

    lax.fori_loop(0, GROUPS, group_body, 0)

    pltpu.sync_copy(out_v, out_hbm.at[pl.ds(base, BPW)])


@jax.jit
def kernel(user_ids, movie_ids, user_emb, movie_emb, user_bias, movie_bias):
    mesh = plsc.VectorSubcoreMesh(core_axis_name="c", subcore_axis_name="s")
    call = pl.kernel(
        _sc_body,
        mesh=mesh,
        compiler_params=pltpu.CompilerParams(
            needs_layout_passes=False, use_tc_tiling_on_sc=False),
        out_type=jax.ShapeDtypeStruct((B,), jnp.float32),
        scratch_types=[
            pltpu.VMEM((BPW,), jnp.int32),
            pltpu.VMEM((BPW,), jnp.int32),
            pltpu.VMEM((BPW, D), jnp.float32),
            pltpu.VMEM((BPW, D), jnp.float32),
            pltpu.VMEM((BPW,), jnp.float32),
            pltpu.VMEM((BPW,), jnp.float32),
            pltpu.VMEM((BPW,), jnp.float32),
            pltpu.SemaphoreType.DMA,
        ],
    )
    return call(
        user_ids.astype(jnp.int32),
        movie_ids.astype(jnp.int32),
        user_emb,
        movie_emb,
        user_bias.reshape(-1),
        movie_bias.reshape(-1),
    )
